# Initial kernel scaffold; baseline (speedup 1.0000x reference)
import jax
import jax.numpy as jnp
from jax.experimental import pallas as pl


def kernel(x, pp_edge_index, pp_edge_attr, pool_edge_index, pool_edge_attr, n_fine, W_root, W_msg, a_edge, b):
    raise NotImplementedError("write your pallas kernel here")


# speedup vs baseline: 1.6288x; 1.6288x over previous
import jax
import jax.numpy as jnp
from jax import lax
from jax.experimental import pallas as pl
from jax.experimental.pallas import tpu as pltpu
from jax.experimental.pallas import tpu_sc as plsc

NCOARSE = 10000
NFINE = 40000
DMODEL = 128
EPP_PAD = 327680
EPOOL_PAD = 163840
NC, NS = 2, 16
NDUM = 496


def _bcast_lane(v16, e):
    idx = jnp.full((16, 1), e, dtype=jnp.int32)
    return lax.gather(
        v16, idx,
        lax.GatherDimensionNumbers(
            offset_dims=(), collapsed_slice_dims=(0,), start_index_map=(0,)),
        slice_sizes=(1,),
        mode=lax.GatherScatterMode.PROMISE_IN_BOUNDS)


def _zero_rows0(rows2):
    def row(i, _):
        for j in range(8):
            rows2[0, i, pl.ds(j * 16, 16)] = jnp.zeros((16,), jnp.float32)
        return 0
    lax.fori_loop(0, 128, row, 0)


def _scale_rows(rows2, buf, wv, rr):
    def g_loop(g, _):
        w16 = wv[rr, pl.ds(g * 16, 16)]

        def e_loop(e, _):
            we = _bcast_lane(w16, e)
            rw = g * 16 + e
            for j in range(8):
                sl = pl.ds(j * 16, 16)
                rows2[buf, rw, sl] = rows2[buf, rw, sl] * we
            return 0
        lax.fori_loop(0, 16, e_loop, 0, unroll=2)
        return 0
    lax.fori_loop(0, 8, g_loop, 0)


def _agg_body(x_hbm, src2d, dst2d, w2d, agg2, agg_sp, idx_s, idx_d, wv, rows2,
              gs0, gs1, ss0, ss1):
    c = lax.axis_index("c")
    s = lax.axis_index("s")
    wid = c * NS + s
    gsems = [gs0, gs1]
    ssems = [ss0, ss1]

    _zero_rows0(rows2)
    for i in range(4):
        pltpu.sync_copy(rows2.at[0], agg_sp.at[pl.ds(s * 624 + i * 128, 128)])
    pltpu.sync_copy(rows2.at[0, pl.ds(0, 112)],
                    agg_sp.at[pl.ds(s * 624 + 512, 112)])

    @pl.when(s == 15)
    def _():
        pltpu.sync_copy(rows2.at[0, pl.ds(0, 16)], agg_sp.at[pl.ds(9984, 16)])
    plsc.subcore_barrier()

    base = wid * 80

    def block(bi, _):
        r0 = base + bi * 16
        pltpu.sync_copy(src2d.at[pl.ds(r0, 16)], idx_s)
        pltpu.sync_copy(dst2d.at[pl.ds(r0, 16)], idx_d)
        pltpu.sync_copy(w2d.at[pl.ds(r0, 16)], wv)

        gd = {0: pltpu.async_copy(x_hbm.at[idx_s.at[0]], rows2.at[0], gs0)}
        sd = {}
        for rr in range(16):
            buf = rr & 1
            if rr < 15:
                if rr >= 1:
                    sd[rr - 1].wait()
                nb = (rr + 1) & 1
                gd[rr + 1] = pltpu.async_copy(
                    x_hbm.at[idx_s.at[rr + 1]], rows2.at[nb], gsems[nb])
            gd[rr].wait()
            _scale_rows(rows2, buf, wv, rr)
            sd[rr] = pltpu.async_copy(rows2.at[buf], agg_sp.at[idx_d.at[rr]],
                                      ssems[buf], add=True)
        sd[14].wait()
        sd[15].wait()
        return 0

    lax.fori_loop(0, 5, block, 0)
    plsc.subcore_barrier()
    pltpu.sync_copy(agg_sp.at[pl.ds(s * 624, 624)],
                    agg2.at[c, pl.ds(s * 624, 624)])

    @pl.when(s == 15)
    def _():
        pltpu.sync_copy(agg_sp.at[pl.ds(9984, 16)],
                        agg2.at[c, pl.ds(9984, 16)])


def _run_agg(x, src2d, dst2d, w2d):
    mesh = plsc.VectorSubcoreMesh(core_axis_name="c", subcore_axis_name="s",
                                  num_cores=NC, num_subcores=NS)
    f = pl.kernel(
        _agg_body,
        out_type=jax.ShapeDtypeStruct((NC, NCOARSE, DMODEL), jnp.float32),
        mesh=mesh,
        scratch_types=[
            pltpu.VMEM_SHARED((NCOARSE, DMODEL), jnp.float32),
            pltpu.VMEM((16, 128), jnp.int32),
            pltpu.VMEM((16, 128), jnp.int32),
            pltpu.VMEM((16, 128), jnp.float32),
            pltpu.VMEM((2, 128, DMODEL), jnp.float32),
            pltpu.SemaphoreType.DMA,
            pltpu.SemaphoreType.DMA,
            pltpu.SemaphoreType.DMA,
            pltpu.SemaphoreType.DMA,
        ],
    )
    return f(x, src2d, dst2d, w2d)


def _unpool_body(h_hbm, psrc2d, pdst2d, pw2d, num2, den2, num_sp, den_sp,
                 idx_s, idx_d, idxr2, wv, rows2, zbd, gs0, gs1, ss0, ss1, dsm):
    c = lax.axis_index("c")
    s = lax.axis_index("s")
    gsems = [gs0, gs1]
    ssems = [ss0, ss1]

    def zrow(i, _):
        zbd[pl.ds(i * 16, 16)] = jnp.zeros((16,), jnp.float32)
        return 0
    lax.fori_loop(0, 64, zrow, 0)

    base_e = s * 80

    for p in range(2):
        r_rng = c * 2 + p
        lo = r_rng * 10000

        _zero_rows0(rows2)
        for i in range(5):
            pltpu.sync_copy(rows2.at[0],
                            num_sp.at[pl.ds(s * 656 + i * 128, 128)])
        pltpu.sync_copy(rows2.at[0, pl.ds(0, 16)],
                        num_sp.at[pl.ds(s * 656 + 640, 16)])
        if p == 0:
            @pl.when(s < 8)
            def _():
                for i in range(5):
                    pltpu.sync_copy(zbd.at[pl.ds(0, 1000)],
                                    den_sp.at[pl.ds(s * 5000 + i * 1000, 1000)])
        plsc.subcore_barrier()

        def block(bi, _):
            r0 = base_e + bi * 16
            pltpu.sync_copy(psrc2d.at[pl.ds(r0, 16)], idx_s)
            pltpu.sync_copy(pdst2d.at[pl.ds(r0, 16)], idx_d)
            pltpu.sync_copy(pw2d.at[pl.ds(r0, 16)], wv)
            den_cond = (bi & 1) == c

            gd = {0: pltpu.async_copy(h_hbm.at[idx_s.at[0]], rows2.at[0], gs0)}
            sd = {}
            dd = []
            for rr in range(16):
                buf = rr & 1
                if rr < 15:
                    if rr >= 1:
                        sd[rr - 1].wait()
                    nb = (rr + 1) & 1
                    gd[rr + 1] = pltpu.async_copy(
                        h_hbm.at[idx_s.at[rr + 1]], rows2.at[nb], gsems[nb])
                gd[rr].wait()

                for g in range(8):
                    sl = pl.ds(g * 16, 16)
                    v = idx_d[rr, sl]
                    vl = v - lo
                    inb = jnp.logical_and(vl >= 0, vl < 10000)
                    dm = 10000 + (v & 255)
                    idxr2[buf, sl] = jnp.where(inb, vl, dm)

                _scale_rows(rows2, buf, wv, rr)
                sd[rr] = pltpu.async_copy(rows2.at[buf],
                                          num_sp.at[idxr2.at[buf]],
                                          ssems[buf], add=True)
                if p == 0:
                    @pl.when(den_cond)
                    def _(rr=rr):
                        dd.append(pltpu.async_copy(
                            wv.at[rr], den_sp.at[idx_d.at[rr]], dsm, add=True))
            sd[14].wait()
            sd[15].wait()
            if p == 0:
                @pl.when(den_cond)
                def _():
                    for d in dd:
                        d.wait()
            return 0

        lax.fori_loop(0, 5, block, 0)
        plsc.subcore_barrier()

        pltpu.sync_copy(num_sp.at[pl.ds(s * 624, 624)],
                        num2.at[r_rng, pl.ds(s * 624, 624)])

        @pl.when(s == 15)
        def _():
            pltpu.sync_copy(num_sp.at[pl.ds(9984, 16)],
                            num2.at[r_rng, pl.ds(9984, 16)])
        if p == 0:
            @pl.when(s < 8)
            def _():
                def dout(i, _):
                    o = s * 5000 + i * 1000
                    pltpu.sync_copy(den_sp.at[pl.ds(o, 1000)],
                                    zbd.at[pl.ds(0, 1000)])
                    pltpu.sync_copy(zbd.at[pl.ds(0, 1000)],
                                    den2.at[pl.ds(c * NFINE + o, 1000)])
                    return 0
                lax.fori_loop(0, 5, dout, 0)
        plsc.subcore_barrier()


def _run_unpool(h, psrc2d, pdst2d, pw2d):
    mesh = plsc.VectorSubcoreMesh(core_axis_name="c", subcore_axis_name="s",
                                  num_cores=NC, num_subcores=NS)
    f = pl.kernel(
        _unpool_body,
        out_type=[
            jax.ShapeDtypeStruct((4, NCOARSE, DMODEL), jnp.float32),
            jax.ShapeDtypeStruct((NC * NFINE,), jnp.float32),
        ],
        mesh=mesh,
        scratch_types=[
            pltpu.VMEM_SHARED((NCOARSE + NDUM, DMODEL), jnp.float32),
            pltpu.VMEM_SHARED((NFINE,), jnp.float32),
            pltpu.VMEM((16, 128), jnp.int32),
            pltpu.VMEM((16, 128), jnp.int32),
            pltpu.VMEM((2, 128), jnp.int32),
            pltpu.VMEM((16, 128), jnp.float32),
            pltpu.VMEM((2, 128, DMODEL), jnp.float32),
            pltpu.VMEM((1024,), jnp.float32),
            pltpu.SemaphoreType.DMA,
            pltpu.SemaphoreType.DMA,
            pltpu.SemaphoreType.DMA,
            pltpu.SemaphoreType.DMA,
            pltpu.SemaphoreType.DMA,
        ],
    )
    return f(h, psrc2d, pdst2d, pw2d)


def _we_kernel(attr_ref, amat_ref, o_ref):
    o_ref[...] = jnp.dot(attr_ref[...], amat_ref[...],
                         preferred_element_type=jnp.float32,
                         precision=lax.Precision.HIGHEST)


def _mm_kernel(x_ref, agg_ref, wr_ref, wm_ref, b_ref, h_ref):
    acc = agg_ref[0] + agg_ref[1]
    h = (jnp.dot(x_ref[...], wr_ref[...], preferred_element_type=jnp.float32,
                 precision=lax.Precision.HIGHEST)
         + jnp.dot(acc, wm_ref[...], preferred_element_type=jnp.float32,
                   precision=lax.Precision.HIGHEST)
         + b_ref[...])
    h_ref[...] = jnp.maximum(h, 0.0)


def _div_kernel(num_ref, den_ref, o_ref):
    d = den_ref[0] + den_ref[1]
    o_ref[...] = num_ref[...] / jnp.maximum(d, 1e-8)


def kernel(x, pp_edge_index, pp_edge_attr, pool_edge_index, pool_edge_attr,
           n_fine, W_root, W_msg, a_edge, b):
    del n_fine
    f32 = jnp.float32
    i32 = jnp.int32

    epp = pp_edge_index.shape[1]
    epool = pool_edge_index.shape[1]

    src2d = jnp.pad(pp_edge_index[0].astype(i32), (0, EPP_PAD - epp)).reshape(
        EPP_PAD // 128, 128)
    dst2d = jnp.pad(pp_edge_index[1].astype(i32), (0, EPP_PAD - epp)).reshape(
        EPP_PAD // 128, 128)
    attr_rs = jnp.pad(pp_edge_attr.astype(f32),
                      ((0, EPP_PAD - epp), (0, 0))).reshape(EPP_PAD // 32, 128)
    psrc2d = jnp.pad(pool_edge_index[0].astype(i32),
                     (0, EPOOL_PAD - epool)).reshape(EPOOL_PAD // 128, 128)
    pdst2d = jnp.pad(pool_edge_index[1].astype(i32),
                     (0, EPOOL_PAD - epool)).reshape(EPOOL_PAD // 128, 128)
    pw2d = jnp.pad(pool_edge_attr[:, 0].astype(f32),
                   (0, EPOOL_PAD - epool)).reshape(EPOOL_PAD // 128, 128)

    amat = jnp.kron(jnp.eye(32, dtype=f32), a_edge.astype(f32)[:, None])

    w_rs = pl.pallas_call(
        _we_kernel,
        grid=(EPP_PAD // 32 // 512,),
        in_specs=[
            pl.BlockSpec((512, 128), lambda m: (m, 0)),
            pl.BlockSpec((128, 32), lambda m: (0, 0)),
        ],
        out_specs=pl.BlockSpec((512, 32), lambda m: (m, 0)),
        out_shape=jax.ShapeDtypeStruct((EPP_PAD // 32, 32), f32),
    )(attr_rs, amat)
    w2d = w_rs.reshape(EPP_PAD // 128, 128)

    agg2 = _run_agg(x, src2d, dst2d, w2d)

    h = pl.pallas_call(
        _mm_kernel,
        grid=(10,),
        in_specs=[
            pl.BlockSpec((1000, DMODEL), lambda m: (m, 0)),
            pl.BlockSpec((NC, 1000, DMODEL), lambda m: (0, m, 0)),
            pl.BlockSpec((DMODEL, DMODEL), lambda m: (0, 0)),
            pl.BlockSpec((DMODEL, DMODEL), lambda m: (0, 0)),
            pl.BlockSpec((1, DMODEL), lambda m: (0, 0)),
        ],
        out_specs=pl.BlockSpec((1000, DMODEL), lambda m: (m, 0)),
        out_shape=jax.ShapeDtypeStruct((NCOARSE, DMODEL), f32),
    )(x, agg2, W_root, W_msg, b.reshape(1, DMODEL))

    num2, den2 = _run_unpool(h, psrc2d, pdst2d, pw2d)

    out = pl.pallas_call(
        _div_kernel,
        grid=(40,),
        in_specs=[
            pl.BlockSpec((1000, DMODEL), lambda m: (m, 0)),
            pl.BlockSpec((NC, 1000, 1), lambda m: (0, m, 0)),
        ],
        out_specs=pl.BlockSpec((1000, DMODEL), lambda m: (m, 0)),
        out_shape=jax.ShapeDtypeStruct((NFINE, DMODEL), f32),
    )(num2.reshape(NFINE, DMODEL), den2.reshape(NC, NFINE, 1))
    return out
